# Initial kernel scaffold; baseline (speedup 1.0000x reference)
import jax
import jax.numpy as jnp
from jax.experimental import pallas as pl


def kernel(node, edge_index, edge_attr, batch_ptr, params):
    raise NotImplementedError("write your pallas kernel here")


# speedup vs baseline: 2.5004x; 2.5004x over previous
import functools

import jax
import jax.numpy as jnp
from jax import lax
from jax.experimental import pallas as pl
from jax.experimental.pallas import tpu as pltpu
from jax.experimental.pallas import tpu_sc as plsc

_N = 10000
_NP = 10240
_E = 160000
_EP = 163840
_D = 256
_HALF = 128
_L = 16
_NTILES = 16
_EPT = _EP // _NTILES
_KB = 128
_NB = _EPT // _KB
_STRIPE = _NP // _NTILES
_CHUNK = 128
_NCHUNK = _STRIPE // _CHUNK


def _sc_segment_sum(node2, src, dst, w):
    mesh = plsc.VectorSubcoreMesh(core_axis_name="c", subcore_axis_name="s")

    @functools.partial(
        pl.kernel,
        mesh=mesh,
        out_type=jax.ShapeDtypeStruct((2, _NP, _HALF), jnp.float32),
        scratch_types=[
            pltpu.VMEM_SHARED((_NP, _HALF), jnp.float32),
            pltpu.VMEM((_KB,), jnp.int32),
            pltpu.VMEM((_KB,), jnp.int32),
            pltpu.VMEM((_KB,), jnp.float32),
            pltpu.VMEM((_KB, _HALF), jnp.float32),
        ],
    )
    def k(node2_h, src_h, dst_h, w_h, out_h, acc, srcb, dstb, wb, rows):
        c = lax.axis_index("c")
        s = lax.axis_index("s")
        zero16 = jnp.zeros((_L,), jnp.float32)

        def zrow(r, _):
            for j in range(_HALF // _L):
                rows[r, pl.ds(j * _L, _L)] = zero16
            return 0
        lax.fori_loop(0, _KB, zrow, 0, unroll=2)

        for kk in range(_NCHUNK):
            r0 = s * _STRIPE + kk * _CHUNK
            pltpu.sync_copy(rows, acc.at[pl.ds(r0, _CHUNK)])
        plsc.subcore_barrier()

        def body(b, _):
            off = s * _EPT + b * _KB
            pltpu.sync_copy(src_h.at[pl.ds(off, _KB)], srcb)
            pltpu.sync_copy(dst_h.at[pl.ds(off, _KB)], dstb)
            pltpu.sync_copy(w_h.at[pl.ds(off, _KB)], wb)
            for t in range(_KB // _L):
                sv = srcb[pl.ds(t * _L, _L)]
                srcb[pl.ds(t * _L, _L)] = sv * 2 + c
            pltpu.sync_copy(node2_h.at[srcb], rows)

            for t in range(_KB // _L):
                wchunk = wb[pl.ds(t * _L, _L)]
                for i in range(_L):
                    wv = jnp.full((_L,), wchunk[i])
                    r = t * _L + i
                    for j in range(_HALF // _L):
                        rows[r, pl.ds(j * _L, _L)] = rows[r, pl.ds(j * _L, _L)] * wv

            pltpu.sync_copy(rows, acc.at[dstb], add=True)
            return 0
        lax.fori_loop(0, _NB, body, 0)
        plsc.subcore_barrier()

        for kk in range(_NCHUNK):
            r0 = s * _STRIPE + kk * _CHUNK
            pltpu.sync_copy(acc.at[pl.ds(r0, _CHUNK)], rows)
            pltpu.sync_copy(rows, out_h.at[c, pl.ds(r0, _CHUNK)])

    return k(node2, src, dst, w)


def _sc_degree(dst):
    mesh = plsc.VectorSubcoreMesh(core_axis_name="c", subcore_axis_name="s")
    half_rows = _NP // 2
    tile_rows = half_rows // _NTILES
    wb_chunk = 64
    n_wb = tile_rows // wb_chunk

    @functools.partial(
        pl.kernel,
        mesh=mesh,
        out_type=jax.ShapeDtypeStruct((_NP, _HALF), jnp.float32),
        scratch_types=[
            pltpu.VMEM_SHARED((_NP, _HALF), jnp.float32),
            pltpu.VMEM((_KB,), jnp.int32),
            pltpu.VMEM((_KB, _HALF), jnp.float32),
        ],
    )
    def k(dst_h, out_h, cnt, dstb, ones):
        c = lax.axis_index("c")
        s = lax.axis_index("s")
        zero16 = jnp.zeros((_L,), jnp.float32)
        one16 = jnp.ones((_L,), jnp.float32)

        def zrow(r, _):
            for j in range(_HALF // _L):
                ones[r, pl.ds(j * _L, _L)] = zero16
            return 0
        lax.fori_loop(0, _KB, zrow, 0, unroll=2)

        for kk in range(_NCHUNK):
            r0 = s * _STRIPE + kk * _CHUNK
            pltpu.sync_copy(ones, cnt.at[pl.ds(r0, _CHUNK)])

        def orow(r, _):
            for j in range(_HALF // _L):
                ones[r, pl.ds(j * _L, _L)] = one16
            return 0
        lax.fori_loop(0, _KB, orow, 0, unroll=2)
        plsc.subcore_barrier()

        def body(b, _):
            off = s * _EPT + b * _KB
            pltpu.sync_copy(dst_h.at[pl.ds(off, _KB)], dstb)
            pltpu.sync_copy(ones, cnt.at[dstb], add=True)
            return 0
        lax.fori_loop(0, _NB, body, 0)
        plsc.subcore_barrier()

        for kk in range(n_wb):
            r0 = c * half_rows + s * tile_rows + kk * wb_chunk
            pltpu.sync_copy(cnt.at[pl.ds(r0, wb_chunk)],
                            ones.at[pl.ds(0, wb_chunk)])
            pltpu.sync_copy(ones.at[pl.ds(0, wb_chunk)],
                            out_h.at[pl.ds(r0, wb_chunk)])

    return k(dst)


def _layer_norm(h, g, b, eps=1e-5):
    mu = jnp.mean(h, axis=-1, keepdims=True)
    var = jnp.mean((h - mu) ** 2, axis=-1, keepdims=True)
    return (h - mu) * lax.rsqrt(var + eps) * g + b


def _dense_body(x_ref, m_ref, c_ref, wrel_ref, wroot_ref, w1_ref, w2_ref, p_ref, o_ref):
    x = x_ref[...]
    m = m_ref[...]
    ssum = jnp.concatenate([m[0], m[1]], axis=-1)
    deg = jnp.maximum(c_ref[...][:, 0:1], 1.0)
    mean = ssum / deg
    P = p_ref[...]
    brel, lng, lnb = P[0], P[1], P[2]
    h4 = (jnp.dot(mean, wrel_ref[...], preferred_element_type=jnp.float32)
          + jnp.dot(x, wroot_ref[...], preferred_element_type=jnp.float32)
          + brel[None, :])
    acc = jnp.zeros_like(x)
    for i in range(4):
        h = x + h4[:, _D * i:_D * (i + 1)]
        y = _layer_norm(h, lng[_D * i:_D * (i + 1)], lnb[_D * i:_D * (i + 1)])
        acc = acc + jnp.maximum(y, 0.0)
    t = jnp.dot(acc, w1_ref[...], preferred_element_type=jnp.float32) + P[3, 0:_D][None, :]
    t = jnp.maximum(_layer_norm(t, P[3, _D:2 * _D], P[3, 2 * _D:3 * _D]), 0.0)
    t = jnp.dot(t, w2_ref[...], preferred_element_type=jnp.float32) + P[3, 3 * _D:4 * _D][None, :]
    o_ref[...] = jnp.maximum(_layer_norm(t, P[4, 0:_D], P[4, _D:2 * _D]), 0.0)


def _dense(node, sums2, cnt, wrel, wroot, w1, w2, pvec):
    R = 1000
    grid = (_N // R,)
    return pl.pallas_call(
        _dense_body,
        grid=grid,
        in_specs=[
            pl.BlockSpec((R, _D), lambda i: (i, 0)),
            pl.BlockSpec((2, R, _HALF), lambda i: (0, i, 0)),
            pl.BlockSpec((R, _HALF), lambda i: (i, 0)),
            pl.BlockSpec((_D, 4 * _D), lambda i: (0, 0)),
            pl.BlockSpec((_D, 4 * _D), lambda i: (0, 0)),
            pl.BlockSpec((_D, _D), lambda i: (0, 0)),
            pl.BlockSpec((_D, _D), lambda i: (0, 0)),
            pl.BlockSpec((8, 4 * _D), lambda i: (0, 0)),
        ],
        out_specs=pl.BlockSpec((R, _D), lambda i: (i, 0)),
        out_shape=jax.ShapeDtypeStruct((_N, _D), jnp.float32),
    )(node, sums2, cnt, wrel, wroot, w1, w2, pvec)


def kernel(node, edge_index, edge_attr, batch_ptr, params):
    del batch_ptr
    pad = _EP - _E
    src = jnp.concatenate([edge_index[0], jnp.zeros((pad,), jnp.int32)])
    dst = jnp.concatenate([edge_index[1],
                           jnp.full((pad,), _NP - 1, jnp.int32)])
    w = jnp.concatenate([edge_attr, jnp.zeros((pad,), jnp.float32)])
    node2 = node.reshape(2 * _N, _HALF)

    sums2 = _sc_segment_sum(node2, src, dst, w)
    cnt = _sc_degree(dst)

    wrel = jnp.concatenate([params[f"W_rel_{i}"].T for i in range(4)], axis=1)
    wroot = jnp.concatenate([params[f"W_root_{i}"].T for i in range(4)], axis=1)
    w1 = params["mlp_W1"].T
    w2 = params["mlp_W2"].T
    row0 = jnp.concatenate([params[f"b_rel_{i}"] for i in range(4)])
    row1 = jnp.concatenate([params[f"ln_g_{i}"] for i in range(4)])
    row2 = jnp.concatenate([params[f"ln_b_{i}"] for i in range(4)])
    row3 = jnp.concatenate([params["mlp_b1"], params["mlp_ln1_g"],
                            params["mlp_ln1_b"], params["mlp_b2"]])
    row4 = jnp.concatenate([params["mlp_ln2_g"], params["mlp_ln2_b"],
                            jnp.zeros((2 * _D,), jnp.float32)])
    zrow = jnp.zeros((4 * _D,), jnp.float32)
    pvec = jnp.stack([row0, row1, row2, row3, row4, zrow, zrow, zrow])

    return _dense(node, sums2, cnt, wrel, wroot, w1, w2, pvec)


# speedup vs baseline: 3.5292x; 1.4115x over previous
import functools

import jax
import jax.numpy as jnp
from jax import lax
from jax.experimental import pallas as pl
from jax.experimental.pallas import tpu as pltpu
from jax.experimental.pallas import tpu_sc as plsc

_N = 10000
_NP = 10240
_E = 160000
_EP = 163840
_D = 256
_HALF = 128
_L = 16
_NTILES = 16
_EPT = _EP // _NTILES
_KB = 128
_NB = _EPT // _KB
_SBB = 10
_SB = _SBB * _KB
_NSB = _NB // _SBB
_STRIPE = _NP // _NTILES
_CHUNK = 128
_NCHUNK = _STRIPE // _CHUNK


def _sc_segment_sum(node2, src, dst, w):
    mesh = plsc.VectorSubcoreMesh(core_axis_name="c", subcore_axis_name="s")

    @functools.partial(
        pl.kernel,
        mesh=mesh,
        out_type=jax.ShapeDtypeStruct((2, _NP, _HALF), jnp.float32),
        scratch_types=[
            pltpu.VMEM_SHARED((_NP, _HALF), jnp.float32),
            pltpu.VMEM((_SB,), jnp.int32),
            pltpu.VMEM((2, _KB), jnp.int32),
            pltpu.VMEM((_SB,), jnp.float32),
            pltpu.VMEM((_KB, _HALF), jnp.float32),
            pltpu.VMEM((_KB, _HALF), jnp.float32),
            pltpu.SemaphoreType.DMA,
            pltpu.SemaphoreType.DMA,
        ],
    )
    def k(node2_h, src_h, dst_h, w_h, out_h,
          acc, srcb, dstb, wb, rows0, rows1, gsem, ssem):
        c = lax.axis_index("c")
        s = lax.axis_index("s")
        zero16 = jnp.zeros((_L,), jnp.float32)
        rows = (rows0, rows1)

        def zrow(r, _):
            for j in range(_HALF // _L):
                rows0[r, pl.ds(j * _L, _L)] = zero16
            return 0
        lax.fori_loop(0, _KB, zrow, 0, unroll=2)

        for kk in range(_NCHUNK):
            r0 = s * _STRIPE + kk * _CHUNK
            pltpu.sync_copy(rows0, acc.at[pl.ds(r0, _CHUNK)])
        plsc.subcore_barrier()

        def scale(buf, w_off):
            def chunk(t, _):
                wchunk = wb[pl.ds(w_off + t * _L, _L)]
                for i in range(_L):
                    wv = jnp.full((_L,), wchunk[i])
                    r = t * _L + i
                    for j in range(_HALF // _L):
                        buf[r, pl.ds(j * _L, _L)] = buf[r, pl.ds(j * _L, _L)] * wv
                return 0
            lax.fori_loop(0, _KB // _L, chunk, 0)

        def super_body(sb, _):
            off = s * _EPT + sb * _SB
            pltpu.sync_copy(src_h.at[pl.ds(off, _SB)], srcb)
            pltpu.sync_copy(w_h.at[pl.ds(off, _SB)], wb)

            def idxt(t, _):
                sv = srcb[pl.ds(t * _L, _L)]
                srcb[pl.ds(t * _L, _L)] = sv * 2 + c
                return 0
            lax.fori_loop(0, _SB // _L, idxt, 0, unroll=4)

            g = pltpu.async_copy(node2_h.at[srcb.at[pl.ds(0, _KB)]],
                                 rows0, gsem)
            pltpu.sync_copy(dst_h.at[pl.ds(off, _KB)], dstb.at[0])
            sc_prev = None
            for j in range(_SBB):
                p = j % 2
                g.wait()
                if j + 1 < _SBB:
                    if sc_prev is not None:
                        sc_prev.wait()
                    g = pltpu.async_copy(
                        node2_h.at[srcb.at[pl.ds((j + 1) * _KB, _KB)]],
                        rows[1 - p], gsem)
                    pltpu.sync_copy(dst_h.at[pl.ds(off + (j + 1) * _KB, _KB)],
                                    dstb.at[1 - p])
                scale(rows[p], j * _KB)
                sc = pltpu.async_copy(rows[p], acc.at[dstb.at[p]],
                                      ssem, add=True)
                if sc_prev is not None and j + 1 >= _SBB:
                    sc_prev.wait()
                sc_prev = sc
            sc_prev.wait()
            return 0
        lax.fori_loop(0, _NSB, super_body, 0)
        plsc.subcore_barrier()

        for kk in range(_NCHUNK):
            r0 = s * _STRIPE + kk * _CHUNK
            pltpu.sync_copy(acc.at[pl.ds(r0, _CHUNK)], rows0)
            pltpu.sync_copy(rows0, out_h.at[c, pl.ds(r0, _CHUNK)])

    return k(node2, src, dst, w)


def _sc_degree(dst):
    mesh = plsc.VectorSubcoreMesh(core_axis_name="c", subcore_axis_name="s")
    ept2 = _EPT // 2
    nb2 = ept2 // _KB

    @functools.partial(
        pl.kernel,
        mesh=mesh,
        out_type=jax.ShapeDtypeStruct((2, _NP, _HALF), jnp.float32),
        scratch_types=[
            pltpu.VMEM_SHARED((_NP, _HALF), jnp.float32),
            pltpu.VMEM((_KB,), jnp.int32),
            pltpu.VMEM((_KB,), jnp.int32),
            pltpu.VMEM((_KB, _HALF), jnp.float32),
            pltpu.SemaphoreType.DMA,
        ],
    )
    def k(dst_h, out_h, cnt, dstb0, dstb1, ones, sem):
        c = lax.axis_index("c")
        s = lax.axis_index("s")
        zero16 = jnp.zeros((_L,), jnp.float32)
        one16 = jnp.ones((_L,), jnp.float32)
        dstb = (dstb0, dstb1)

        def zrow(r, _):
            for j in range(_HALF // _L):
                ones[r, pl.ds(j * _L, _L)] = zero16
            return 0
        lax.fori_loop(0, _KB, zrow, 0, unroll=2)

        for kk in range(_NCHUNK):
            r0 = s * _STRIPE + kk * _CHUNK
            pltpu.sync_copy(ones, cnt.at[pl.ds(r0, _CHUNK)])

        def orow(r, _):
            for j in range(_HALF // _L):
                ones[r, pl.ds(j * _L, _L)] = one16
            return 0
        lax.fori_loop(0, _KB, orow, 0, unroll=2)
        plsc.subcore_barrier()

        base = (c * _NTILES + s) * ept2
        g = pltpu.async_copy(dst_h.at[pl.ds(base, _KB)], dstb0, sem)
        g.wait()
        for b in range(nb2):
            p = b % 2
            if b + 1 < nb2:
                g2 = pltpu.async_copy(
                    dst_h.at[pl.ds(base + (b + 1) * _KB, _KB)],
                    dstb[1 - p], sem)
            pltpu.sync_copy(ones, cnt.at[dstb[p]], add=True)
            if b + 1 < nb2:
                g2.wait()
        plsc.subcore_barrier()

        for kk in range(_NCHUNK):
            r0 = s * _STRIPE + kk * _CHUNK
            pltpu.sync_copy(cnt.at[pl.ds(r0, _CHUNK)], ones)
            pltpu.sync_copy(ones, out_h.at[c, pl.ds(r0, _CHUNK)])

    return k(dst)


def _layer_norm(h, g, b, eps=1e-5):
    mu = jnp.mean(h, axis=-1, keepdims=True)
    var = jnp.mean((h - mu) ** 2, axis=-1, keepdims=True)
    return (h - mu) * lax.rsqrt(var + eps) * g + b


def _dense_body(x_ref, m_ref, c_ref, wrel_ref, wroot_ref, w1_ref, w2_ref, p_ref, o_ref):
    x = x_ref[...]
    m = m_ref[...]
    ssum = jnp.concatenate([m[0], m[1]], axis=-1)
    cc = c_ref[...]
    deg = jnp.maximum(cc[0][:, 0:1] + cc[1][:, 0:1], 1.0)
    mean = ssum / deg
    P = p_ref[...]
    brel, lng, lnb = P[0], P[1], P[2]
    h4 = (jnp.dot(mean, wrel_ref[...], preferred_element_type=jnp.float32)
          + jnp.dot(x, wroot_ref[...], preferred_element_type=jnp.float32)
          + brel[None, :])
    acc = jnp.zeros_like(x)
    for i in range(4):
        h = x + h4[:, _D * i:_D * (i + 1)]
        y = _layer_norm(h, lng[_D * i:_D * (i + 1)], lnb[_D * i:_D * (i + 1)])
        acc = acc + jnp.maximum(y, 0.0)
    t = jnp.dot(acc, w1_ref[...], preferred_element_type=jnp.float32) + P[3, 0:_D][None, :]
    t = jnp.maximum(_layer_norm(t, P[3, _D:2 * _D], P[3, 2 * _D:3 * _D]), 0.0)
    t = jnp.dot(t, w2_ref[...], preferred_element_type=jnp.float32) + P[3, 3 * _D:4 * _D][None, :]
    o_ref[...] = jnp.maximum(_layer_norm(t, P[4, 0:_D], P[4, _D:2 * _D]), 0.0)


def _dense(node, sums2, cnt2, wrel, wroot, w1, w2, pvec):
    R = 1000
    grid = (_N // R,)
    return pl.pallas_call(
        _dense_body,
        grid=grid,
        in_specs=[
            pl.BlockSpec((R, _D), lambda i: (i, 0)),
            pl.BlockSpec((2, R, _HALF), lambda i: (0, i, 0)),
            pl.BlockSpec((2, R, _HALF), lambda i: (0, i, 0)),
            pl.BlockSpec((_D, 4 * _D), lambda i: (0, 0)),
            pl.BlockSpec((_D, 4 * _D), lambda i: (0, 0)),
            pl.BlockSpec((_D, _D), lambda i: (0, 0)),
            pl.BlockSpec((_D, _D), lambda i: (0, 0)),
            pl.BlockSpec((8, 4 * _D), lambda i: (0, 0)),
        ],
        out_specs=pl.BlockSpec((R, _D), lambda i: (i, 0)),
        out_shape=jax.ShapeDtypeStruct((_N, _D), jnp.float32),
    )(node, sums2, cnt2, wrel, wroot, w1, w2, pvec)


def kernel(node, edge_index, edge_attr, batch_ptr, params):
    del batch_ptr
    pad = _EP - _E
    src = jnp.concatenate([edge_index[0], jnp.zeros((pad,), jnp.int32)])
    dst = jnp.concatenate([edge_index[1],
                           jnp.full((pad,), _NP - 1, jnp.int32)])
    w = jnp.concatenate([edge_attr, jnp.zeros((pad,), jnp.float32)])
    node2 = node.reshape(2 * _N, _HALF)

    sums2 = _sc_segment_sum(node2, src, dst, w)
    cnt2 = _sc_degree(dst)

    wrel = jnp.concatenate([params[f"W_rel_{i}"].T for i in range(4)], axis=1)
    wroot = jnp.concatenate([params[f"W_root_{i}"].T for i in range(4)], axis=1)
    w1 = params["mlp_W1"].T
    w2 = params["mlp_W2"].T
    row0 = jnp.concatenate([params[f"b_rel_{i}"] for i in range(4)])
    row1 = jnp.concatenate([params[f"ln_g_{i}"] for i in range(4)])
    row2 = jnp.concatenate([params[f"ln_b_{i}"] for i in range(4)])
    row3 = jnp.concatenate([params["mlp_b1"], params["mlp_ln1_g"],
                            params["mlp_ln1_b"], params["mlp_b2"]])
    row4 = jnp.concatenate([params["mlp_ln2_g"], params["mlp_ln2_b"],
                            jnp.zeros((2 * _D,), jnp.float32)])
    zrow = jnp.zeros((4 * _D,), jnp.float32)
    pvec = jnp.stack([row0, row1, row2, row3, row4, zrow, zrow, zrow])

    return _dense(node, sums2, cnt2, wrel, wroot, w1, w2, pvec)


# speedup vs baseline: 3.7676x; 1.0676x over previous
import functools

import jax
import jax.numpy as jnp
from jax import lax
from jax.experimental import pallas as pl
from jax.experimental.pallas import tpu as pltpu
from jax.experimental.pallas import tpu_sc as plsc

_N = 10000
_NP = 10240
_E = 160000
_EP = 163840
_D = 256
_HALF = 128
_L = 16
_NTILES = 16
_EPT = _EP // _NTILES
_KB = 128
_NB = _EPT // _KB
_SBB = 16
_SB = _SBB * _KB
_NSB = _NB // _SBB
_STRIPE = _NP // _NTILES
_CHUNK = 128
_NCHUNK = _STRIPE // _CHUNK


def _sc_segment_sum(node2, src, dst3, w):
    mesh = plsc.VectorSubcoreMesh(core_axis_name="c", subcore_axis_name="s")

    @functools.partial(
        pl.kernel,
        mesh=mesh,
        out_type=jax.ShapeDtypeStruct((2, _NP, _HALF), jnp.float32),
        scratch_types=[
            pltpu.VMEM_SHARED((_NP, _HALF), jnp.float32),
            pltpu.VMEM((_SB,), jnp.int32),
            pltpu.VMEM((_SBB, _KB), jnp.int32),
            pltpu.VMEM((_SB,), jnp.float32),
            pltpu.VMEM((_KB, _HALF), jnp.float32),
            pltpu.VMEM((_KB, _HALF), jnp.float32),
            pltpu.SemaphoreType.DMA,
            pltpu.SemaphoreType.DMA,
        ],
    )
    def k(node2_h, src_h, dst3_h, w_h, out_h,
          acc, srcb, dstb, wb, rows0, rows1, gsem, ssem):
        c = lax.axis_index("c")
        s = lax.axis_index("s")
        zero16 = jnp.zeros((_L,), jnp.float32)
        rows = (rows0, rows1)

        def zrow(r, _):
            for j in range(_HALF // _L):
                rows0[r, pl.ds(j * _L, _L)] = zero16
            return 0
        lax.fori_loop(0, _KB, zrow, 0, unroll=2)

        for kk in range(_NCHUNK):
            r0 = s * _STRIPE + kk * _CHUNK
            pltpu.sync_copy(rows0, acc.at[pl.ds(r0, _CHUNK)])
        plsc.subcore_barrier()

        def scale(buf, w_off):
            def chunk(t, _):
                wchunk = wb[pl.ds(w_off + t * _L, _L)]
                for i in range(_L):
                    wv = jnp.full((_L,), wchunk[i])
                    r = t * _L + i
                    for j in range(_HALF // _L):
                        buf[r, pl.ds(j * _L, _L)] = buf[r, pl.ds(j * _L, _L)] * wv
                return 0
            lax.fori_loop(0, _KB // _L, chunk, 0)

        def super_body(sb, _):
            off = s * _EPT + sb * _SB
            pltpu.sync_copy(src_h.at[pl.ds(off, _SB)], srcb)
            pltpu.sync_copy(w_h.at[pl.ds(off, _SB)], wb)
            pltpu.sync_copy(dst3_h.at[pl.ds(s * _NB + sb * _SBB, _SBB)], dstb)

            def idxt(t, _):
                sv = srcb[pl.ds(t * _L, _L)]
                srcb[pl.ds(t * _L, _L)] = sv * 2 + c
                return 0
            lax.fori_loop(0, _SB // _L, idxt, 0, unroll=4)

            g = pltpu.async_copy(node2_h.at[srcb.at[pl.ds(0, _KB)]],
                                 rows0, gsem)
            sc_prev = None
            for j in range(_SBB):
                p = j % 2
                g.wait()
                if j + 1 < _SBB:
                    if sc_prev is not None:
                        sc_prev.wait()
                    g = pltpu.async_copy(
                        node2_h.at[srcb.at[pl.ds((j + 1) * _KB, _KB)]],
                        rows[1 - p], gsem)
                scale(rows[p], j * _KB)
                sc = pltpu.async_copy(rows[p], acc.at[dstb.at[j]],
                                      ssem, add=True)
                if sc_prev is not None and j + 1 >= _SBB:
                    sc_prev.wait()
                sc_prev = sc
            sc_prev.wait()
            return 0
        lax.fori_loop(0, _NSB, super_body, 0)
        plsc.subcore_barrier()

        for kk in range(_NCHUNK):
            r0 = s * _STRIPE + kk * _CHUNK
            pltpu.sync_copy(acc.at[pl.ds(r0, _CHUNK)], rows0)
            pltpu.sync_copy(rows0, out_h.at[c, pl.ds(r0, _CHUNK)])

    return k(node2, src, dst3, w)


def _sc_degree(dst):
    mesh = plsc.VectorSubcoreMesh(core_axis_name="c", subcore_axis_name="s")
    ept2 = _EPT // 2
    nb2 = ept2 // _KB

    @functools.partial(
        pl.kernel,
        mesh=mesh,
        out_type=jax.ShapeDtypeStruct((2, _NP, _HALF), jnp.float32),
        scratch_types=[
            pltpu.VMEM_SHARED((_NP, _HALF), jnp.float32),
            pltpu.VMEM((_KB,), jnp.int32),
            pltpu.VMEM((_KB,), jnp.int32),
            pltpu.VMEM((_KB, _HALF), jnp.float32),
            pltpu.SemaphoreType.DMA,
        ],
    )
    def k(dst_h, out_h, cnt, dstb0, dstb1, ones, sem):
        c = lax.axis_index("c")
        s = lax.axis_index("s")
        zero16 = jnp.zeros((_L,), jnp.float32)
        one16 = jnp.ones((_L,), jnp.float32)
        dstb = (dstb0, dstb1)

        def zrow(r, _):
            for j in range(_HALF // _L):
                ones[r, pl.ds(j * _L, _L)] = zero16
            return 0
        lax.fori_loop(0, _KB, zrow, 0, unroll=2)

        for kk in range(_NCHUNK):
            r0 = s * _STRIPE + kk * _CHUNK
            pltpu.sync_copy(ones, cnt.at[pl.ds(r0, _CHUNK)])

        def orow(r, _):
            for j in range(_HALF // _L):
                ones[r, pl.ds(j * _L, _L)] = one16
            return 0
        lax.fori_loop(0, _KB, orow, 0, unroll=2)
        plsc.subcore_barrier()

        base = (c * _NTILES + s) * ept2
        g = pltpu.async_copy(dst_h.at[pl.ds(base, _KB)], dstb0, sem)
        g.wait()
        for b in range(nb2):
            p = b % 2
            if b + 1 < nb2:
                g2 = pltpu.async_copy(
                    dst_h.at[pl.ds(base + (b + 1) * _KB, _KB)],
                    dstb[1 - p], sem)
            pltpu.sync_copy(ones, cnt.at[dstb[p]], add=True)
            if b + 1 < nb2:
                g2.wait()
        plsc.subcore_barrier()

        for kk in range(_NCHUNK):
            r0 = s * _STRIPE + kk * _CHUNK
            pltpu.sync_copy(cnt.at[pl.ds(r0, _CHUNK)], ones)
            pltpu.sync_copy(ones, out_h.at[c, pl.ds(r0, _CHUNK)])

    return k(dst)


def _layer_norm(h, g, b, eps=1e-5):
    mu = jnp.mean(h, axis=-1, keepdims=True)
    var = jnp.mean((h - mu) ** 2, axis=-1, keepdims=True)
    return (h - mu) * lax.rsqrt(var + eps) * g + b


def _dense_body(x_ref, m_ref, c_ref, wrel_ref, wroot_ref, w1_ref, w2_ref, p_ref, o_ref):
    x = x_ref[...]
    m = m_ref[...]
    ssum = jnp.concatenate([m[0], m[1]], axis=-1)
    cc = c_ref[...]
    deg = jnp.maximum(cc[0][:, 0:1] + cc[1][:, 0:1], 1.0)
    mean = ssum / deg
    P = p_ref[...]
    brel, lng, lnb = P[0], P[1], P[2]
    h4 = (jnp.dot(mean, wrel_ref[...], preferred_element_type=jnp.float32)
          + jnp.dot(x, wroot_ref[...], preferred_element_type=jnp.float32)
          + brel[None, :])
    acc = jnp.zeros_like(x)
    for i in range(4):
        h = x + h4[:, _D * i:_D * (i + 1)]
        y = _layer_norm(h, lng[_D * i:_D * (i + 1)], lnb[_D * i:_D * (i + 1)])
        acc = acc + jnp.maximum(y, 0.0)
    t = jnp.dot(acc, w1_ref[...], preferred_element_type=jnp.float32) + P[3, 0:_D][None, :]
    t = jnp.maximum(_layer_norm(t, P[3, _D:2 * _D], P[3, 2 * _D:3 * _D]), 0.0)
    t = jnp.dot(t, w2_ref[...], preferred_element_type=jnp.float32) + P[3, 3 * _D:4 * _D][None, :]
    o_ref[...] = jnp.maximum(_layer_norm(t, P[4, 0:_D], P[4, _D:2 * _D]), 0.0)


def _dense(node, sums2, cnt2, wrel, wroot, w1, w2, pvec):
    R = 1000
    grid = (_N // R,)
    return pl.pallas_call(
        _dense_body,
        grid=grid,
        in_specs=[
            pl.BlockSpec((R, _D), lambda i: (i, 0)),
            pl.BlockSpec((2, R, _HALF), lambda i: (0, i, 0)),
            pl.BlockSpec((2, R, _HALF), lambda i: (0, i, 0)),
            pl.BlockSpec((_D, 4 * _D), lambda i: (0, 0)),
            pl.BlockSpec((_D, 4 * _D), lambda i: (0, 0)),
            pl.BlockSpec((_D, _D), lambda i: (0, 0)),
            pl.BlockSpec((_D, _D), lambda i: (0, 0)),
            pl.BlockSpec((8, 4 * _D), lambda i: (0, 0)),
        ],
        out_specs=pl.BlockSpec((R, _D), lambda i: (i, 0)),
        out_shape=jax.ShapeDtypeStruct((_N, _D), jnp.float32),
    )(node, sums2, cnt2, wrel, wroot, w1, w2, pvec)


def kernel(node, edge_index, edge_attr, batch_ptr, params):
    del batch_ptr
    pad = _EP - _E
    src = jnp.concatenate([edge_index[0], jnp.zeros((pad,), jnp.int32)])
    dst = jnp.concatenate([edge_index[1],
                           jnp.full((pad,), _NP - 1, jnp.int32)])
    w = jnp.concatenate([edge_attr, jnp.zeros((pad,), jnp.float32)])
    node2 = node.reshape(2 * _N, _HALF)

    sums2 = _sc_segment_sum(node2, src, dst.reshape(_EP // _KB, _KB), w)
    cnt2 = _sc_degree(dst)

    wrel = jnp.concatenate([params[f"W_rel_{i}"].T for i in range(4)], axis=1)
    wroot = jnp.concatenate([params[f"W_root_{i}"].T for i in range(4)], axis=1)
    w1 = params["mlp_W1"].T
    w2 = params["mlp_W2"].T
    row0 = jnp.concatenate([params[f"b_rel_{i}"] for i in range(4)])
    row1 = jnp.concatenate([params[f"ln_g_{i}"] for i in range(4)])
    row2 = jnp.concatenate([params[f"ln_b_{i}"] for i in range(4)])
    row3 = jnp.concatenate([params["mlp_b1"], params["mlp_ln1_g"],
                            params["mlp_ln1_b"], params["mlp_b2"]])
    row4 = jnp.concatenate([params["mlp_ln2_g"], params["mlp_ln2_b"],
                            jnp.zeros((2 * _D,), jnp.float32)])
    zrow = jnp.zeros((4 * _D,), jnp.float32)
    pvec = jnp.stack([row0, row1, row2, row3, row4, zrow, zrow, zrow])

    return _dense(node, sums2, cnt2, wrel, wroot, w1, w2, pvec)
